# baseline (speedup 1.0000x reference)
import functools

import jax
import jax.numpy as jnp
from jax import lax
from jax.experimental import pallas as pl
from jax.experimental.pallas import tpu as pltpu
from jax.experimental.pallas import tpu_sc as plsc

N = 10000
R = 8
E = 160000
E_PN = 20000
H = 256
NB = 4
NSEG = N * R

NC = 2
NS = 16
NW = NC * NS

WIN = 128
NWINDOWS = NSEG // WIN
NWS = 640
GB = 128
HV = H // 16

SCORE_PER_W = 1280
E_SC = SCORE_PER_W * NW

_INTERPRET = False


def _wid():
    return lax.axis_index("s") * NC + lax.axis_index("c")


def _segsum_body(h_hbm, src_hbm, comb_hbm, ws_hbm, s_hbm, cnt_hbm,
                 ws_v, idx0, idx1, cmb0, cmb1, gb0, gb1, stag, cnt_v, sem):
    wid = _wid()
    lane0 = lax.iota(jnp.int32, 16) == 0
    one0 = jnp.where(lane0, 1, 0)
    pltpu.sync_copy(ws_hbm, ws_v)
    bufs = ((idx0, cmb0, gb0), (idx1, cmb1, gb1))

    def window(jj, _):
        j = wid + jj * NW

        @pl.when(j < NWINDOWS)
        def _():
            wsv = ws_v[pl.ds(j, 16)]
            e0 = wsv[0]
            e1 = wsv[1]
            c0 = j * WIN

            def zrow(i, _):
                for k in range(HV):
                    stag[i, pl.ds(k * 16, 16)] = jnp.zeros((16,), jnp.float32)
                return 0
            lax.fori_loop(0, WIN, zrow, 0)
            for k in range((WIN + 16) // 16):
                cnt_v[pl.ds(k * 16, 16)] = jnp.zeros((16,), jnp.int32)

            base0 = (e0 // 8) * 8
            nb = (e1 - base0 + GB - 1) // GB

            def fetch(k, p):
                fidx, fcmb, fgb = bufs[p]
                base = base0 + k * GB
                pltpu.sync_copy(src_hbm.at[pl.ds(base, GB)], fidx)
                pltpu.sync_copy(comb_hbm.at[pl.ds(base, GB)],
                                fcmb.at[pl.ds(0, GB)])
                pltpu.async_copy(h_hbm.at[fidx], fgb, sem.at[p])

            @pl.when(nb > 0)
            def _():
                fetch(0, 0)

            def accum(k, p):
                bidx, bcmb, bgb = bufs[p]
                base = base0 + k * GB
                pltpu.make_async_copy(h_hbm.at[bidx], bgb, sem.at[p]).wait()
                lo = jnp.maximum(e0 - base, 0)
                hi = jnp.minimum(e1 - base, GB)
                lanes = lax.iota(jnp.int32, 16)

                def group(g, _):
                    gbase = g * 16
                    cv = bcmb[pl.ds(gbase, 16)] - c0
                    ev = lanes + gbase
                    valid = (ev >= lo) & (ev < hi)
                    rowv = jnp.where(valid, cv, WIN)
                    for l in range(16):
                        row = rowv[l]
                        plsc.addupdate(cnt_v.at[pl.ds(row, 16)], one0)
                        for k2 in range(HV):
                            sl = pl.ds(k2 * 16, 16)
                            plsc.addupdate(stag.at[row, sl],
                                           bgb[gbase + l, sl])
                    return 0
                lax.fori_loop(0, GB // 16, group, 0)

            def pair(kk, _):
                for par in range(2):
                    k = 2 * kk + par

                    @pl.when(k < nb)
                    def _():
                        @pl.when(k + 1 < nb)
                        def _():
                            fetch(k + 1, 1 - par)
                        accum(k, par)
                return 0

            lax.fori_loop(0, (nb + 1) // 2, pair, 0)

            pltpu.sync_copy(stag.at[pl.ds(0, WIN)], s_hbm.at[pl.ds(c0, WIN)])
            pltpu.sync_copy(cnt_v.at[pl.ds(0, WIN)],
                            cnt_hbm.at[pl.ds(c0, WIN)])

        return 0

    lax.fori_loop(0, (NWINDOWS + NW - 1) // NW, window, 0)


def _segsum(h, sorted_src, sorted_comb, win_starts):
    mesh = plsc.VectorSubcoreMesh(core_axis_name="c", subcore_axis_name="s", num_cores=NC, num_subcores=NS)
    f = pl.kernel(
        _segsum_body,
        out_type=(
            jax.ShapeDtypeStruct((NSEG, H), jnp.float32),
            jax.ShapeDtypeStruct((NSEG,), jnp.int32),
        ),
        mesh=mesh,
        scratch_types=[
            pltpu.VMEM((NWS,), jnp.int32),
            pltpu.VMEM((GB,), jnp.int32),
            pltpu.VMEM((GB,), jnp.int32),
            pltpu.VMEM((GB + 16,), jnp.int32),
            pltpu.VMEM((GB + 16,), jnp.int32),
            pltpu.VMEM((GB, H), jnp.float32),
            pltpu.VMEM((GB, H), jnp.float32),
            pltpu.VMEM((WIN + 8, H), jnp.float32),
            pltpu.VMEM((WIN + 32,), jnp.int32),
            pltpu.SemaphoreType.DMA((2,)),
        ],
        interpret=_INTERPRET,
    )
    return f(h, sorted_src, sorted_comb, win_starts)


def _distmult_body(emb_hbm, ui_hbm, vi_hbm, et_hbm, wrel_hbm, out_hbm,
                   wrel_v, ui_v, vi_v, et_v, ubuf, vbuf, obuf, sem):
    wid = _wid()
    pltpu.sync_copy(wrel_hbm, wrel_v)
    base_w = wid * SCORE_PER_W

    def batch(k, _):
        base = base_w + k * GB
        pltpu.sync_copy(ui_hbm.at[pl.ds(base, GB)], ui_v)
        pltpu.sync_copy(vi_hbm.at[pl.ds(base, GB)], vi_v)
        pltpu.sync_copy(et_hbm.at[pl.ds(base, GB)], et_v.at[pl.ds(0, GB)])
        cu = pltpu.async_copy(emb_hbm.at[ui_v], ubuf, sem.at[0])
        cv = pltpu.async_copy(emb_hbm.at[vi_v], vbuf, sem.at[1])
        cu.wait()
        cv.wait()

        def edge(i, _):
            et = et_v[pl.ds(i, 16)][0]
            acc = jnp.zeros((16,), jnp.float32)
            for k2 in range(HV):
                sl = pl.ds(k2 * 16, 16)
                acc = acc + ubuf[i, sl] * wrel_v[et, sl] * vbuf[i, sl]
            obuf[i, pl.ds(0, 16)] = acc
            return 0
        lax.fori_loop(0, GB, edge, 0)
        pltpu.sync_copy(obuf, out_hbm.at[pl.ds(base, GB)])
        return 0

    lax.fori_loop(0, SCORE_PER_W // GB, batch, 0)


def _distmult_sc(emb, ui, vi, et, w_rel):
    mesh = plsc.VectorSubcoreMesh(core_axis_name="c", subcore_axis_name="s", num_cores=NC, num_subcores=NS)
    f = pl.kernel(
        _distmult_body,
        out_type=jax.ShapeDtypeStruct((E_SC, 16), jnp.float32),
        mesh=mesh,
        scratch_types=[
            pltpu.VMEM((R, H), jnp.float32),
            pltpu.VMEM((GB,), jnp.int32),
            pltpu.VMEM((GB,), jnp.int32),
            pltpu.VMEM((GB + 16,), jnp.int32),
            pltpu.VMEM((GB, H), jnp.float32),
            pltpu.VMEM((GB, H), jnp.float32),
            pltpu.VMEM((GB, 16), jnp.float32),
            pltpu.SemaphoreType.DMA((2,)),
        ],
        interpret=_INTERPRET,
    )
    return f(emb, ui, vi, et, w_rel)


_BN = 1000


def _score_reduce_body(x_ref, o_ref):
    o_ref[...] = jnp.sum(x_ref[...], axis=1)


def _score_reduce(x16):
    return pl.pallas_call(
        _score_reduce_body,
        out_shape=jax.ShapeDtypeStruct((E_SC,), jnp.float32),
        interpret=_INTERPRET,
    )(x16)


def _mlp_body(x_ref, w_ref, b_ref, o_ref):
    acc = jnp.dot(x_ref[...], w_ref[...], preferred_element_type=jnp.float32)
    o_ref[...] = jnp.maximum(acc + b_ref[...], 0.0)


def _mlp(x, W, b):
    grid = (N // _BN,)
    return pl.pallas_call(
        _mlp_body,
        grid=grid,
        in_specs=[
            pl.BlockSpec((_BN, H), lambda i: (i, 0)),
            pl.BlockSpec((H, H), lambda i: (0, 0)),
            pl.BlockSpec((1, H), lambda i: (0, 0)),
        ],
        out_specs=pl.BlockSpec((_BN, H), lambda i: (i, 0)),
        out_shape=jax.ShapeDtypeStruct((N, H), jnp.float32),
        interpret=_INTERPRET,
    )(x, W, b.reshape(1, H))


def _combine_body(relu, s_ref, cnt_ref, h_ref, a_ref, v_ref, ws_ref, b_ref,
                  o_ref):
    norm = 1.0 / jnp.maximum(cnt_ref[...], 1.0)
    acc = jnp.dot(h_ref[...], ws_ref[...],
                  preferred_element_type=jnp.float32)
    for b in range(NB):
        t = jnp.zeros((_BN, H), jnp.float32)
        for r in range(R):
            t = t + (a_ref[r, b] * norm[:, r])[:, None] * s_ref[:, r, :]
        acc = acc + jnp.dot(t, v_ref[b], preferred_element_type=jnp.float32)
    acc = acc + b_ref[...]
    if relu:
        acc = jnp.maximum(acc, 0.0)
    o_ref[...] = acc


def _combine(s3, cntf, h, A_pad, V, Ws, b, relu):
    grid = (N // _BN,)
    return pl.pallas_call(
        functools.partial(_combine_body, relu),
        grid=grid,
        in_specs=[
            pl.BlockSpec((_BN, R, H), lambda i: (i, 0, 0)),
            pl.BlockSpec((_BN, R), lambda i: (i, 0)),
            pl.BlockSpec((_BN, H), lambda i: (i, 0)),
            pl.BlockSpec((R, 128), lambda i: (0, 0)),
            pl.BlockSpec((NB, H, H), lambda i: (0, 0, 0)),
            pl.BlockSpec((H, H), lambda i: (0, 0)),
            pl.BlockSpec((1, H), lambda i: (0, 0)),
        ],
        out_specs=pl.BlockSpec((_BN, H), lambda i: (i, 0)),
        out_shape=jax.ShapeDtypeStruct((N, H), jnp.float32),
        interpret=_INTERPRET,
    )(s3, cntf, h, A_pad, V, Ws, b.reshape(1, H))


def kernel(x, edge_index, edge_type, pos_edge_index, pos_etype,
           neg_edge_index, neg_etype, W_ft, b_ft, V1, A1, Ws1, b1,
           V2, A2, Ws2, b2, w_rel):
    src, dst = edge_index[0], edge_index[1]
    comb = dst * R + edge_type

    perm = jnp.argsort(comb)
    sorted_comb = comb[perm]
    sorted_src = src[perm]
    pad = GB + 8
    pad_iota = jnp.arange(pad, dtype=jnp.int32)
    sorted_src_p = jnp.concatenate([sorted_src, pad_iota % N])
    sorted_comb_p = jnp.concatenate(
        [sorted_comb, jnp.full((pad,), NSEG - 1, jnp.int32)])
    win_starts = jnp.searchsorted(
        sorted_comb, jnp.arange(0, NSEG + 1, WIN, dtype=jnp.int32)
    ).astype(jnp.int32)
    win_starts = jnp.concatenate(
        [win_starts, jnp.full((NWS - NWINDOWS - 1,), E, jnp.int32)])

    A1p = jnp.zeros((R, 128), jnp.float32).at[:, :NB].set(A1)
    A2p = jnp.zeros((R, 128), jnp.float32).at[:, :NB].set(A2)

    h = _mlp(x, W_ft, b_ft)

    def layer(h, A_pad, V, Ws, b, relu):
        s, cnt = _segsum(h, sorted_src_p, sorted_comb_p, win_starts)
        s3 = s.reshape(N, R, H)
        cntf = cnt.reshape(N, R).astype(jnp.float32)
        return _combine(s3, cntf, h, A_pad, V, Ws, b, relu)

    h = layer(h, A1p, V1, Ws1, b1, True)
    emb = layer(h, A2p, V2, Ws2, b2, False)

    pad_s = E_SC - 2 * E_PN
    pi = jnp.arange(pad_s, dtype=jnp.int32)
    ui = jnp.concatenate([pos_edge_index[0], neg_edge_index[0], pi % N])
    vi = jnp.concatenate([pos_edge_index[1], neg_edge_index[1], pi % N])
    et = jnp.concatenate([pos_etype, neg_etype, pi % R])
    scores16 = _distmult_sc(emb, ui, vi, et, w_rel)
    scores = _score_reduce(scores16)
    return (scores[:E_PN], scores[E_PN:2 * E_PN])


# speedup vs baseline: 1.0603x; 1.0603x over previous
import functools

import jax
import jax.numpy as jnp
from jax import lax
from jax.experimental import pallas as pl
from jax.experimental.pallas import tpu as pltpu
from jax.experimental.pallas import tpu_sc as plsc

N = 10000
R = 8
E = 160000
E_PN = 20000
H = 256
NB = 4
NSEG = N * R

NC = 2
NS = 16
NW = NC * NS

WIN = 128
NWINDOWS = NSEG // WIN
NWS = 656
GB = 128
HV = H // 16

SCORE_PER_W = 1280
E_SC = SCORE_PER_W * NW

_INTERPRET = False


def _wid():
    return lax.axis_index("s") * NC + lax.axis_index("c")


def _segsum_body(h_hbm, src_hbm, comb_hbm, ws_hbm, s_hbm, cnt_hbm,
                 ws_v, idx0, idx1, cmb0, cmb1, gb0, gb1, stag, cnt_v, sem):
    wid = _wid()
    lane0 = lax.iota(jnp.int32, 16) == 0
    one0 = jnp.where(lane0, 1, 0)
    pltpu.sync_copy(ws_hbm, ws_v)
    bufs = ((idx0, cmb0, gb0), (idx1, cmb1, gb1))

    def fetch_at(base, p):
        fidx, fcmb, fgb = bufs[p]
        pltpu.sync_copy(src_hbm.at[pl.ds(base, GB)], fidx)
        pltpu.sync_copy(comb_hbm.at[pl.ds(base, GB)],
                        fcmb.at[pl.ds(0, GB)])
        pltpu.async_copy(h_hbm.at[fidx], fgb, sem.at[p])

    def win_meta(j):
        wsv = ws_v[pl.ds(j, 16)]
        e0 = wsv[0]
        e1 = wsv[1]
        base0 = (e0 // 8) * 8
        nbw = jnp.maximum((e1 - base0 + GB - 1) // GB, 1)
        return e0, e1, base0, nbw

    e0f, _, base0f, _ = win_meta(wid)
    fetch_at(base0f, 0)

    def window(jj, par_in):
        j = wid + jj * NW
        e0, e1, base0, nbw = win_meta(j)
        c0 = j * WIN

        def run(sp):
            def zrow(i, _):
                for k in range(HV):
                    stag[i, pl.ds(k * 16, 16)] = jnp.zeros((16,), jnp.float32)
                return 0
            lax.fori_loop(0, WIN, zrow, 0)
            for k in range((WIN + 16) // 16):
                cnt_v[pl.ds(k * 16, 16)] = jnp.zeros((16,), jnp.int32)

            def accum(k, b):
                bidx, bcmb, bgb = bufs[b]
                base = base0 + k * GB
                pltpu.make_async_copy(h_hbm.at[bidx], bgb, sem.at[b]).wait()
                lo = jnp.maximum(e0 - base, 0)
                hi = jnp.minimum(e1 - base, GB)

                def edge(i, _):
                    row = bcmb[pl.ds(i, 16)][0] - c0
                    plsc.addupdate(cnt_v.at[pl.ds(row, 16)], one0)
                    for k2 in range(HV):
                        sl = pl.ds(k2 * 16, 16)
                        plsc.addupdate(stag.at[row, sl], bgb[i, sl])
                    return 0
                lax.fori_loop(lo, hi, edge, 0)

            def pair(kk, _):
                for par in range(2):
                    k = 2 * kk + par
                    b = sp ^ par

                    @pl.when(k < nbw)
                    def _():
                        @pl.when(k + 1 < nbw)
                        def _():
                            fetch_at(base0 + (k + 1) * GB, 1 - b)

                        @pl.when(k + 1 == nbw)
                        def _():
                            jn = j + NW

                            @pl.when(jn < NWINDOWS)
                            def _():
                                _, _, base0n, _ = win_meta(jn)
                                fetch_at(base0n, 1 - b)
                        accum(k, b)
                return 0

            lax.fori_loop(0, (nbw + 1) // 2, pair, 0)

            pltpu.sync_copy(stag.at[pl.ds(0, WIN)],
                            s_hbm.at[pl.ds(c0, WIN)])
            pltpu.sync_copy(cnt_v.at[pl.ds(0, WIN)],
                            cnt_hbm.at[pl.ds(c0, WIN)])

        @pl.when((j < NWINDOWS) & (par_in == 0))
        def _():
            run(0)

        @pl.when((j < NWINDOWS) & (par_in == 1))
        def _():
            run(1)

        return jnp.where(j < NWINDOWS, par_in ^ (nbw & 1), par_in)

    lax.fori_loop(0, (NWINDOWS + NW - 1) // NW, window, 0)


def _segsum(h, sorted_src, sorted_comb, win_starts):
    mesh = plsc.VectorSubcoreMesh(core_axis_name="c", subcore_axis_name="s", num_cores=NC, num_subcores=NS)
    f = pl.kernel(
        _segsum_body,
        out_type=(
            jax.ShapeDtypeStruct((NSEG, H), jnp.float32),
            jax.ShapeDtypeStruct((NSEG,), jnp.int32),
        ),
        mesh=mesh,
        scratch_types=[
            pltpu.VMEM((NWS,), jnp.int32),
            pltpu.VMEM((GB,), jnp.int32),
            pltpu.VMEM((GB,), jnp.int32),
            pltpu.VMEM((GB + 16,), jnp.int32),
            pltpu.VMEM((GB + 16,), jnp.int32),
            pltpu.VMEM((GB, H), jnp.float32),
            pltpu.VMEM((GB, H), jnp.float32),
            pltpu.VMEM((WIN + 8, H), jnp.float32),
            pltpu.VMEM((WIN + 32,), jnp.int32),
            pltpu.SemaphoreType.DMA((2,)),
        ],
        interpret=_INTERPRET,
    )
    return f(h, sorted_src, sorted_comb, win_starts)


def _distmult_body(emb_hbm, ui_hbm, vi_hbm, et_hbm, wrel_hbm, out_hbm,
                   wrel_v, ui_v, vi_v, et_v, ubuf, vbuf, obuf, sem):
    wid = _wid()
    pltpu.sync_copy(wrel_hbm, wrel_v)
    base_w = wid * SCORE_PER_W

    def batch(k, _):
        base = base_w + k * GB
        pltpu.sync_copy(ui_hbm.at[pl.ds(base, GB)], ui_v)
        pltpu.sync_copy(vi_hbm.at[pl.ds(base, GB)], vi_v)
        pltpu.sync_copy(et_hbm.at[pl.ds(base, GB)], et_v.at[pl.ds(0, GB)])
        cu = pltpu.async_copy(emb_hbm.at[ui_v], ubuf, sem.at[0])
        cv = pltpu.async_copy(emb_hbm.at[vi_v], vbuf, sem.at[1])
        cu.wait()
        cv.wait()

        def edge(i, _):
            et = et_v[pl.ds(i, 16)][0]
            acc = jnp.zeros((16,), jnp.float32)
            for k2 in range(HV):
                sl = pl.ds(k2 * 16, 16)
                acc = acc + ubuf[i, sl] * wrel_v[et, sl] * vbuf[i, sl]
            obuf[i, pl.ds(0, 16)] = acc
            return 0
        lax.fori_loop(0, GB, edge, 0)
        pltpu.sync_copy(obuf, out_hbm.at[pl.ds(base, GB)])
        return 0

    lax.fori_loop(0, SCORE_PER_W // GB, batch, 0)


def _distmult_sc(emb, ui, vi, et, w_rel):
    mesh = plsc.VectorSubcoreMesh(core_axis_name="c", subcore_axis_name="s", num_cores=NC, num_subcores=NS)
    f = pl.kernel(
        _distmult_body,
        out_type=jax.ShapeDtypeStruct((E_SC, 16), jnp.float32),
        mesh=mesh,
        scratch_types=[
            pltpu.VMEM((R, H), jnp.float32),
            pltpu.VMEM((GB,), jnp.int32),
            pltpu.VMEM((GB,), jnp.int32),
            pltpu.VMEM((GB + 16,), jnp.int32),
            pltpu.VMEM((GB, H), jnp.float32),
            pltpu.VMEM((GB, H), jnp.float32),
            pltpu.VMEM((GB, 16), jnp.float32),
            pltpu.SemaphoreType.DMA((2,)),
        ],
        interpret=_INTERPRET,
    )
    return f(emb, ui, vi, et, w_rel)


_BN = 1000


def _score_reduce_body(x_ref, o_ref):
    o_ref[...] = jnp.sum(x_ref[...], axis=1)


def _score_reduce(x16):
    return pl.pallas_call(
        _score_reduce_body,
        out_shape=jax.ShapeDtypeStruct((E_SC,), jnp.float32),
        interpret=_INTERPRET,
    )(x16)


def _mlp_body(x_ref, w_ref, b_ref, o_ref):
    acc = jnp.dot(x_ref[...], w_ref[...], preferred_element_type=jnp.float32)
    o_ref[...] = jnp.maximum(acc + b_ref[...], 0.0)


def _mlp(x, W, b):
    grid = (N // _BN,)
    return pl.pallas_call(
        _mlp_body,
        grid=grid,
        in_specs=[
            pl.BlockSpec((_BN, H), lambda i: (i, 0)),
            pl.BlockSpec((H, H), lambda i: (0, 0)),
            pl.BlockSpec((1, H), lambda i: (0, 0)),
        ],
        out_specs=pl.BlockSpec((_BN, H), lambda i: (i, 0)),
        out_shape=jax.ShapeDtypeStruct((N, H), jnp.float32),
        interpret=_INTERPRET,
    )(x, W, b.reshape(1, H))


def _combine_body(relu, s_ref, cnt_ref, h_ref, a_ref, v_ref, ws_ref, b_ref,
                  o_ref):
    norm = 1.0 / jnp.maximum(cnt_ref[...], 1.0)
    acc = jnp.dot(h_ref[...], ws_ref[...],
                  preferred_element_type=jnp.float32)
    for b in range(NB):
        t = jnp.zeros((_BN, H), jnp.float32)
        for r in range(R):
            t = t + (a_ref[r, b] * norm[:, r])[:, None] * s_ref[:, r, :]
        acc = acc + jnp.dot(t, v_ref[b], preferred_element_type=jnp.float32)
    acc = acc + b_ref[...]
    if relu:
        acc = jnp.maximum(acc, 0.0)
    o_ref[...] = acc


def _combine(s3, cntf, h, A_pad, V, Ws, b, relu):
    grid = (N // _BN,)
    return pl.pallas_call(
        functools.partial(_combine_body, relu),
        grid=grid,
        in_specs=[
            pl.BlockSpec((_BN, R, H), lambda i: (i, 0, 0)),
            pl.BlockSpec((_BN, R), lambda i: (i, 0)),
            pl.BlockSpec((_BN, H), lambda i: (i, 0)),
            pl.BlockSpec((R, 128), lambda i: (0, 0)),
            pl.BlockSpec((NB, H, H), lambda i: (0, 0, 0)),
            pl.BlockSpec((H, H), lambda i: (0, 0)),
            pl.BlockSpec((1, H), lambda i: (0, 0)),
        ],
        out_specs=pl.BlockSpec((_BN, H), lambda i: (i, 0)),
        out_shape=jax.ShapeDtypeStruct((N, H), jnp.float32),
        interpret=_INTERPRET,
    )(s3, cntf, h, A_pad, V, Ws, b.reshape(1, H))


def kernel(x, edge_index, edge_type, pos_edge_index, pos_etype,
           neg_edge_index, neg_etype, W_ft, b_ft, V1, A1, Ws1, b1,
           V2, A2, Ws2, b2, w_rel):
    src, dst = edge_index[0], edge_index[1]
    comb = dst * R + edge_type

    perm = jnp.argsort(comb)
    sorted_comb = comb[perm]
    sorted_src = src[perm]
    pad = GB + 8
    pad_iota = jnp.arange(pad, dtype=jnp.int32)
    sorted_src_p = jnp.concatenate([sorted_src, pad_iota % N])
    sorted_comb_p = jnp.concatenate(
        [sorted_comb, jnp.full((pad,), NSEG - 1, jnp.int32)])
    win_starts = jnp.searchsorted(
        sorted_comb, jnp.arange(0, NSEG + 1, WIN, dtype=jnp.int32)
    ).astype(jnp.int32)
    win_starts = jnp.concatenate(
        [win_starts, jnp.full((NWS - NWINDOWS - 1,), E, jnp.int32)])

    A1p = jnp.zeros((R, 128), jnp.float32).at[:, :NB].set(A1)
    A2p = jnp.zeros((R, 128), jnp.float32).at[:, :NB].set(A2)

    h = _mlp(x, W_ft, b_ft)

    def layer(h, A_pad, V, Ws, b, relu):
        s, cnt = _segsum(h, sorted_src_p, sorted_comb_p, win_starts)
        s3 = s.reshape(N, R, H)
        cntf = cnt.reshape(N, R).astype(jnp.float32)
        return _combine(s3, cntf, h, A_pad, V, Ws, b, relu)

    h = layer(h, A1p, V1, Ws1, b1, True)
    emb = layer(h, A2p, V2, Ws2, b2, False)

    pad_s = E_SC - 2 * E_PN
    pi = jnp.arange(pad_s, dtype=jnp.int32)
    ui = jnp.concatenate([pos_edge_index[0], neg_edge_index[0], pi % N])
    vi = jnp.concatenate([pos_edge_index[1], neg_edge_index[1], pi % N])
    et = jnp.concatenate([pos_etype, neg_etype, pi % R])
    scores16 = _distmult_sc(emb, ui, vi, et, w_rel)
    scores = _score_reduce(scores16)
    return (scores[:E_PN], scores[E_PN:2 * E_PN])


# speedup vs baseline: 1.4367x; 1.3550x over previous
import functools

import jax
import jax.numpy as jnp
from jax import lax
from jax.experimental import pallas as pl
from jax.experimental.pallas import tpu as pltpu
from jax.experimental.pallas import tpu_sc as plsc

N = 10000
R = 8
E = 160000
E_PN = 20000
H = 256
NB = 4
NSEG = N * R

NC = 2
NS = 16
NW = NC * NS

WIN = 128
NWINDOWS = NSEG // WIN
NWS = 656
GB = 128
HV = H // 16

SCORE_PER_W = 1280
E_SC = SCORE_PER_W * NW

_INTERPRET = False


def _wid():
    return lax.axis_index("s") * NC + lax.axis_index("c")


def _segsum_body(h_hbm, src_hbm, comb_hbm, ws_hbm, s_hbm, cnt_hbm,
                 ws_v, idx0, idx1, cmb0, cmb1, gb0, gb1, stag, cnt_v, sem):
    wid = _wid()
    lane0 = lax.iota(jnp.int32, 16) == 0
    one0 = jnp.where(lane0, 1, 0)
    pltpu.sync_copy(ws_hbm, ws_v)
    bufs = ((idx0, cmb0, gb0), (idx1, cmb1, gb1))

    def fetch_at(base, p):
        fidx, fcmb, fgb = bufs[p]
        pltpu.sync_copy(src_hbm.at[pl.ds(base, GB)], fidx)
        pltpu.sync_copy(comb_hbm.at[pl.ds(base, GB)],
                        fcmb.at[pl.ds(0, GB)])
        pltpu.async_copy(h_hbm.at[fidx], fgb, sem.at[p])

    def win_meta(j):
        wsv = ws_v[pl.ds(j, 16)]
        e0 = wsv[0]
        e1 = wsv[1]
        base0 = (e0 // 8) * 8
        nbw = jnp.maximum((e1 - base0 + GB - 1) // GB, 1)
        return e0, e1, base0, nbw

    e0f, _, base0f, _ = win_meta(wid)
    fetch_at(base0f, 0)

    def window(jj, par_in):
        j = wid + jj * NW
        e0, e1, base0, nbw = win_meta(j)
        c0 = j * WIN

        def run(sp):
            def zrow(i, _):
                for k in range(HV):
                    stag[i, pl.ds(k * 16, 16)] = jnp.zeros((16,), jnp.float32)
                return 0
            lax.fori_loop(0, WIN, zrow, 0)
            for k in range((WIN + 16) // 16):
                cnt_v[pl.ds(k * 16, 16)] = jnp.zeros((16,), jnp.int32)

            def accum(k, b):
                bidx, bcmb, bgb = bufs[b]
                base = base0 + k * GB
                pltpu.make_async_copy(h_hbm.at[bidx], bgb, sem.at[b]).wait()
                lo = jnp.maximum(e0 - base, 0)
                hi = jnp.minimum(e1 - base, GB)

                @plsc.parallel_loop(lo, hi, unroll=4)
                def edge(i):
                    row = bcmb[pl.ds(i, 16)][0] - c0
                    plsc.addupdate(cnt_v.at[pl.ds(row, 16)], one0)
                    for k2 in range(HV):
                        sl = pl.ds(k2 * 16, 16)
                        plsc.addupdate(stag.at[row, sl], bgb[i, sl])

            def pair(kk, _):
                for par in range(2):
                    k = 2 * kk + par
                    b = sp ^ par

                    @pl.when(k < nbw)
                    def _():
                        @pl.when(k + 1 < nbw)
                        def _():
                            fetch_at(base0 + (k + 1) * GB, 1 - b)

                        @pl.when(k + 1 == nbw)
                        def _():
                            jn = j + NW

                            @pl.when(jn < NWINDOWS)
                            def _():
                                _, _, base0n, _ = win_meta(jn)
                                fetch_at(base0n, 1 - b)
                        accum(k, b)
                return 0

            lax.fori_loop(0, (nbw + 1) // 2, pair, 0)

            pltpu.sync_copy(stag.at[pl.ds(0, WIN)],
                            s_hbm.at[pl.ds(c0, WIN)])
            pltpu.sync_copy(cnt_v.at[pl.ds(0, WIN)],
                            cnt_hbm.at[pl.ds(c0, WIN)])

        @pl.when((j < NWINDOWS) & (par_in == 0))
        def _():
            run(0)

        @pl.when((j < NWINDOWS) & (par_in == 1))
        def _():
            run(1)

        return jnp.where(j < NWINDOWS, par_in ^ (nbw & 1), par_in)

    lax.fori_loop(0, (NWINDOWS + NW - 1) // NW, window, 0)


def _segsum(h, sorted_src, sorted_comb, win_starts):
    mesh = plsc.VectorSubcoreMesh(core_axis_name="c", subcore_axis_name="s", num_cores=NC, num_subcores=NS)
    f = pl.kernel(
        _segsum_body,
        out_type=(
            jax.ShapeDtypeStruct((NSEG, H), jnp.float32),
            jax.ShapeDtypeStruct((NSEG,), jnp.int32),
        ),
        mesh=mesh,
        scratch_types=[
            pltpu.VMEM((NWS,), jnp.int32),
            pltpu.VMEM((GB,), jnp.int32),
            pltpu.VMEM((GB,), jnp.int32),
            pltpu.VMEM((GB + 16,), jnp.int32),
            pltpu.VMEM((GB + 16,), jnp.int32),
            pltpu.VMEM((GB, H), jnp.float32),
            pltpu.VMEM((GB, H), jnp.float32),
            pltpu.VMEM((WIN + 8, H), jnp.float32),
            pltpu.VMEM((WIN + 32,), jnp.int32),
            pltpu.SemaphoreType.DMA((2,)),
        ],
        interpret=_INTERPRET,
    )
    return f(h, sorted_src, sorted_comb, win_starts)


def _distmult_body(emb_hbm, ui_hbm, vi_hbm, et_hbm, wrel_hbm, out_hbm,
                   wrel_v, ui_v, vi_v, et_v, ubuf, vbuf, obuf, sem):
    wid = _wid()
    pltpu.sync_copy(wrel_hbm, wrel_v)
    base_w = wid * SCORE_PER_W

    def batch(k, _):
        base = base_w + k * GB
        pltpu.sync_copy(ui_hbm.at[pl.ds(base, GB)], ui_v)
        pltpu.sync_copy(vi_hbm.at[pl.ds(base, GB)], vi_v)
        pltpu.sync_copy(et_hbm.at[pl.ds(base, GB)], et_v.at[pl.ds(0, GB)])
        cu = pltpu.async_copy(emb_hbm.at[ui_v], ubuf, sem.at[0])
        cv = pltpu.async_copy(emb_hbm.at[vi_v], vbuf, sem.at[1])
        cu.wait()
        cv.wait()

        def edge(i, _):
            et = et_v[pl.ds(i, 16)][0]
            acc = jnp.zeros((16,), jnp.float32)
            for k2 in range(HV):
                sl = pl.ds(k2 * 16, 16)
                acc = acc + ubuf[i, sl] * wrel_v[et, sl] * vbuf[i, sl]
            obuf[i, pl.ds(0, 16)] = acc
            return 0
        lax.fori_loop(0, GB, edge, 0)
        pltpu.sync_copy(obuf, out_hbm.at[pl.ds(base, GB)])
        return 0

    lax.fori_loop(0, SCORE_PER_W // GB, batch, 0)


def _distmult_sc(emb, ui, vi, et, w_rel):
    mesh = plsc.VectorSubcoreMesh(core_axis_name="c", subcore_axis_name="s", num_cores=NC, num_subcores=NS)
    f = pl.kernel(
        _distmult_body,
        out_type=jax.ShapeDtypeStruct((E_SC, 16), jnp.float32),
        mesh=mesh,
        scratch_types=[
            pltpu.VMEM((R, H), jnp.float32),
            pltpu.VMEM((GB,), jnp.int32),
            pltpu.VMEM((GB,), jnp.int32),
            pltpu.VMEM((GB + 16,), jnp.int32),
            pltpu.VMEM((GB, H), jnp.float32),
            pltpu.VMEM((GB, H), jnp.float32),
            pltpu.VMEM((GB, 16), jnp.float32),
            pltpu.SemaphoreType.DMA((2,)),
        ],
        interpret=_INTERPRET,
    )
    return f(emb, ui, vi, et, w_rel)


_BN = 1000


def _score_reduce_body(x_ref, o_ref):
    o_ref[...] = jnp.sum(x_ref[...], axis=1)


def _score_reduce(x16):
    return pl.pallas_call(
        _score_reduce_body,
        out_shape=jax.ShapeDtypeStruct((E_SC,), jnp.float32),
        interpret=_INTERPRET,
    )(x16)


def _mlp_body(x_ref, w_ref, b_ref, o_ref):
    acc = jnp.dot(x_ref[...], w_ref[...], preferred_element_type=jnp.float32)
    o_ref[...] = jnp.maximum(acc + b_ref[...], 0.0)


def _mlp(x, W, b):
    grid = (N // _BN,)
    return pl.pallas_call(
        _mlp_body,
        grid=grid,
        in_specs=[
            pl.BlockSpec((_BN, H), lambda i: (i, 0)),
            pl.BlockSpec((H, H), lambda i: (0, 0)),
            pl.BlockSpec((1, H), lambda i: (0, 0)),
        ],
        out_specs=pl.BlockSpec((_BN, H), lambda i: (i, 0)),
        out_shape=jax.ShapeDtypeStruct((N, H), jnp.float32),
        interpret=_INTERPRET,
    )(x, W, b.reshape(1, H))


def _combine_body(relu, s_ref, cnt_ref, h_ref, a_ref, v_ref, ws_ref, b_ref,
                  o_ref):
    norm = 1.0 / jnp.maximum(cnt_ref[...], 1.0)
    acc = jnp.dot(h_ref[...], ws_ref[...],
                  preferred_element_type=jnp.float32)
    for b in range(NB):
        t = jnp.zeros((_BN, H), jnp.float32)
        for r in range(R):
            t = t + (a_ref[r, b] * norm[:, r])[:, None] * s_ref[:, r, :]
        acc = acc + jnp.dot(t, v_ref[b], preferred_element_type=jnp.float32)
    acc = acc + b_ref[...]
    if relu:
        acc = jnp.maximum(acc, 0.0)
    o_ref[...] = acc


def _combine(s3, cntf, h, A_pad, V, Ws, b, relu):
    grid = (N // _BN,)
    return pl.pallas_call(
        functools.partial(_combine_body, relu),
        grid=grid,
        in_specs=[
            pl.BlockSpec((_BN, R, H), lambda i: (i, 0, 0)),
            pl.BlockSpec((_BN, R), lambda i: (i, 0)),
            pl.BlockSpec((_BN, H), lambda i: (i, 0)),
            pl.BlockSpec((R, 128), lambda i: (0, 0)),
            pl.BlockSpec((NB, H, H), lambda i: (0, 0, 0)),
            pl.BlockSpec((H, H), lambda i: (0, 0)),
            pl.BlockSpec((1, H), lambda i: (0, 0)),
        ],
        out_specs=pl.BlockSpec((_BN, H), lambda i: (i, 0)),
        out_shape=jax.ShapeDtypeStruct((N, H), jnp.float32),
        interpret=_INTERPRET,
    )(s3, cntf, h, A_pad, V, Ws, b.reshape(1, H))


def kernel(x, edge_index, edge_type, pos_edge_index, pos_etype,
           neg_edge_index, neg_etype, W_ft, b_ft, V1, A1, Ws1, b1,
           V2, A2, Ws2, b2, w_rel):
    src, dst = edge_index[0], edge_index[1]
    comb = dst * R + edge_type

    perm = jnp.argsort(comb)
    sorted_comb = comb[perm]
    sorted_src = src[perm]
    pad = GB + 8
    pad_iota = jnp.arange(pad, dtype=jnp.int32)
    sorted_src_p = jnp.concatenate([sorted_src, pad_iota % N])
    sorted_comb_p = jnp.concatenate(
        [sorted_comb, jnp.full((pad,), NSEG - 1, jnp.int32)])
    win_starts = jnp.searchsorted(
        sorted_comb, jnp.arange(0, NSEG + 1, WIN, dtype=jnp.int32)
    ).astype(jnp.int32)
    win_starts = jnp.concatenate(
        [win_starts, jnp.full((NWS - NWINDOWS - 1,), E, jnp.int32)])

    A1p = jnp.zeros((R, 128), jnp.float32).at[:, :NB].set(A1)
    A2p = jnp.zeros((R, 128), jnp.float32).at[:, :NB].set(A2)

    h = _mlp(x, W_ft, b_ft)

    def layer(h, A_pad, V, Ws, b, relu):
        s, cnt = _segsum(h, sorted_src_p, sorted_comb_p, win_starts)
        s3 = s.reshape(N, R, H)
        cntf = cnt.reshape(N, R).astype(jnp.float32)
        return _combine(s3, cntf, h, A_pad, V, Ws, b, relu)

    h = layer(h, A1p, V1, Ws1, b1, True)
    emb = layer(h, A2p, V2, Ws2, b2, False)

    pad_s = E_SC - 2 * E_PN
    pi = jnp.arange(pad_s, dtype=jnp.int32)
    ui = jnp.concatenate([pos_edge_index[0], neg_edge_index[0], pi % N])
    vi = jnp.concatenate([pos_edge_index[1], neg_edge_index[1], pi % N])
    et = jnp.concatenate([pos_etype, neg_etype, pi % R])
    scores16 = _distmult_sc(emb, ui, vi, et, w_rel)
    scores = _score_reduce(scores16)
    return (scores[:E_PN], scores[E_PN:2 * E_PN])
